# baseline (speedup 1.0000x reference)
import jax
import jax.numpy as jnp
from jax.experimental import pallas as pl
from jax.experimental.pallas import tpu as pltpu

ISIZE = 224
PSIZE = 30
PLANE = ISIZE * ISIZE
TW = 8192
BS = 8


def _place_kernel(pos_ref, x_ref, pf_ref, out_ref):
    g = pl.program_id(0)
    out_ref[...] = x_ref[...]
    for i in range(BS):
        b = g * BS + i
        shift = pos_ref[b, 0] * ISIZE + pos_ref[b, 1]
        off = jnp.minimum((shift // 128) * 128, PLANE - TW)
        off = pl.multiple_of(off, 128)
        tile = pltpu.roll(pf_ref[...], shift - off, axis=1)
        win = x_ref[3 * i:3 * i + 3, pl.ds(off, TW)]
        out_ref[3 * i:3 * i + 3, pl.ds(off, TW)] = win + tile


def kernel(x, patch, pos):
    B = x.shape[0]
    xf = x.reshape(B * 3, PLANE)
    pf = jnp.zeros((3, TW // ISIZE + 1, ISIZE), dtype=patch.dtype)
    pf = jax.lax.dynamic_update_slice(pf, patch[0], (0, 0, 0))
    pf = pf.reshape(3, -1)[:, :TW]
    grid_spec = pltpu.PrefetchScalarGridSpec(
        num_scalar_prefetch=1,
        grid=(B // BS,),
        in_specs=[
            pl.BlockSpec((3 * BS, PLANE), lambda b, pos_ref: (b, 0)),
            pl.BlockSpec((3, TW), lambda b, pos_ref: (0, 0)),
        ],
        out_specs=pl.BlockSpec((3 * BS, PLANE), lambda b, pos_ref: (b, 0)),
    )
    out = pl.pallas_call(
        _place_kernel,
        grid_spec=grid_spec,
        out_shape=jax.ShapeDtypeStruct(xf.shape, x.dtype),
    )(pos, xf, pf)
    return out.reshape(x.shape)
